# baseline (speedup 1.0000x reference)
import functools

import jax
import jax.numpy as jnp
from jax import lax
from jax.experimental import pallas as pl
from jax.experimental.pallas import tpu as pltpu
from jax.experimental.pallas import tpu_sc as plsc

R = 128
N = 32768
NC = 2
NS = 16
L = 16
NW = NC * NS
SC_COLS = 8192
CW = SC_COLS // NW
RB = 32
NRB = R // RB
NG = CW // L
CHAIN = 8


def _block_reduce(buf, col, row0, m_in, i_in):
    chains = []
    for c in range(RB // CHAIN):
        r = c * CHAIN
        m = buf[r, col]
        idx = jnp.full((L,), row0 + r, jnp.int32)
        for i in range(r + 1, r + CHAIN):
            v = buf[i, col]
            p = v > m
            m = jnp.where(p, v, m)
            idx = jnp.where(p, row0 + i, idx)
        chains.append((m, idx))
    while len(chains) > 1:
        nxt = []
        for a in range(0, len(chains), 2):
            (ma, ia), (mb, ib) = chains[a], chains[a + 1]
            p = mb > ma
            nxt.append((jnp.where(p, mb, ma), jnp.where(p, ib, ia)))
        chains = nxt
    m, idx = chains[0]
    if m_in is not None:
        p = m > m_in
        m = jnp.where(p, m, m_in)
        idx = jnp.where(p, idx, i_in)
    return m, idx


def _argmax_body(x_hbm, out_hbm, buf0, buf1, mscr, iscr, sem0, sem1, osem):
    wid = lax.axis_index("s") * NC + lax.axis_index("c")
    base = wid * CW
    bufs = (buf0, buf1)
    sems = (sem0, sem1)

    def fetch(rb):
        return pltpu.async_copy(
            x_hbm.at[pl.ds(rb * RB, RB), pl.ds(base, CW)], bufs[rb % 2],
            sems[rb % 2])

    dma = fetch(0)
    for rb in range(NRB):
        nxt = fetch(rb + 1) if rb + 1 < NRB else None
        dma.wait()
        buf = bufs[rb % 2]
        first = rb == 0

        def gbody(g, _, buf=buf, first=first, rb=rb):
            col = pl.ds(g * L, L)
            if first:
                m_in, i_in = None, None
            else:
                m_in, i_in = mscr[col], iscr[col]
            m, idx = _block_reduce(buf, col, rb * RB, m_in, i_in)
            mscr[col] = m
            iscr[col] = idx
            return 0

        lax.fori_loop(0, NG, gbody, 0)
        dma = nxt

    pltpu.async_copy(iscr, out_hbm.at[pl.ds(base, CW)], osem).wait()


_argmax_sc = functools.partial(
    pl.kernel,
    out_type=jax.ShapeDtypeStruct((SC_COLS,), jnp.int32),
    mesh=plsc.VectorSubcoreMesh(core_axis_name="c", subcore_axis_name="s",
                                num_cores=NC, num_subcores=NS),
    scratch_types=[
        pltpu.VMEM((RB, CW), jnp.float32),
        pltpu.VMEM((RB, CW), jnp.float32),
        pltpu.VMEM((CW,), jnp.float32),
        pltpu.VMEM((CW,), jnp.int32),
        pltpu.SemaphoreType.DMA,
        pltpu.SemaphoreType.DMA,
        pltpu.SemaphoreType.DMA,
    ],
)(_argmax_body)


TC_BLK = 2048


def _tc_body(x_ref, o_ref):
    xb = x_ref[...]
    m = jnp.max(xb, axis=0)
    rowid = lax.broadcasted_iota(jnp.int32, xb.shape, 0)
    cand = jnp.where(xb == m[None, :], rowid, R)
    o_ref[...] = jnp.min(cand, axis=0)


def _argmax_tc(x, start, ncols):
    off = start // TC_BLK
    return pl.pallas_call(
        _tc_body,
        grid=(ncols // TC_BLK,),
        in_specs=[pl.BlockSpec((R, TC_BLK), lambda i, off=off: (0, i + off))],
        out_specs=pl.BlockSpec((TC_BLK,), lambda i: (i,)),
        out_shape=jax.ShapeDtypeStruct((ncols,), jnp.int32),
    )(x)


@jax.jit
def kernel(x):
    sc = _argmax_sc(x)
    tc = _argmax_tc(x, SC_COLS, N - SC_COLS)
    return jnp.concatenate([sc, tc]).astype(jnp.int64)


# speedup vs baseline: 2.8489x; 2.8489x over previous
import functools

import jax
import jax.numpy as jnp
from jax import lax
from jax.experimental import pallas as pl
from jax.experimental.pallas import tpu as pltpu
from jax.experimental.pallas import tpu_sc as plsc

R = 128
N = 32768
NC = 2
NS = 16
L = 16
NW = NC * NS
SC_COLS = 0
CW = max(SC_COLS, NW) // NW
RB = 32
NRB = R // RB
NG = CW // L
CHAIN = 8


def _block_reduce(buf, col, row0, m_in, i_in):
    chains = []
    for c in range(RB // CHAIN):
        r = c * CHAIN
        m = buf[r, col]
        idx = jnp.full((L,), row0 + r, jnp.int32)
        for i in range(r + 1, r + CHAIN):
            v = buf[i, col]
            p = v > m
            m = jnp.where(p, v, m)
            idx = jnp.where(p, row0 + i, idx)
        chains.append((m, idx))
    while len(chains) > 1:
        nxt = []
        for a in range(0, len(chains), 2):
            (ma, ia), (mb, ib) = chains[a], chains[a + 1]
            p = mb > ma
            nxt.append((jnp.where(p, mb, ma), jnp.where(p, ib, ia)))
        chains = nxt
    m, idx = chains[0]
    if m_in is not None:
        p = m > m_in
        m = jnp.where(p, m, m_in)
        idx = jnp.where(p, idx, i_in)
    return m, idx


def _argmax_body(x_hbm, out_hbm, buf0, buf1, mscr, iscr, sem0, sem1, osem):
    wid = lax.axis_index("s") * NC + lax.axis_index("c")
    base = wid * CW
    bufs = (buf0, buf1)
    sems = (sem0, sem1)

    def fetch(rb):
        return pltpu.async_copy(
            x_hbm.at[pl.ds(rb * RB, RB), pl.ds(base, CW)], bufs[rb % 2],
            sems[rb % 2])

    dma = fetch(0)
    for rb in range(NRB):
        nxt = fetch(rb + 1) if rb + 1 < NRB else None
        dma.wait()
        buf = bufs[rb % 2]
        first = rb == 0

        def gbody(g, _, buf=buf, first=first, rb=rb):
            col = pl.ds(g * L, L)
            if first:
                m_in, i_in = None, None
            else:
                m_in, i_in = mscr[col], iscr[col]
            m, idx = _block_reduce(buf, col, rb * RB, m_in, i_in)
            mscr[col] = m
            iscr[col] = idx
            return 0

        lax.fori_loop(0, NG, gbody, 0)
        dma = nxt

    pltpu.async_copy(iscr, out_hbm.at[pl.ds(base, CW)], osem).wait()


if SC_COLS:
    _argmax_sc = functools.partial(
        pl.kernel,
        out_type=jax.ShapeDtypeStruct((SC_COLS,), jnp.int32),
        mesh=plsc.VectorSubcoreMesh(core_axis_name="c", subcore_axis_name="s",
                                    num_cores=NC, num_subcores=NS),
        scratch_types=[
            pltpu.VMEM((RB, CW), jnp.float32),
            pltpu.VMEM((RB, CW), jnp.float32),
            pltpu.VMEM((CW,), jnp.float32),
            pltpu.VMEM((CW,), jnp.int32),
            pltpu.SemaphoreType.DMA,
            pltpu.SemaphoreType.DMA,
            pltpu.SemaphoreType.DMA,
        ],
    )(_argmax_body)


TC_BLK = 4096


def _tc_body(x_ref, o_ref):
    xb = x_ref[...]
    m = jnp.max(xb, axis=0)
    rowid = lax.broadcasted_iota(jnp.int32, xb.shape, 0)
    cand = jnp.where(xb == m[None, :], rowid, R)
    o_ref[...] = jnp.min(cand, axis=0)


def _argmax_tc(x, start, ncols):
    off = start // TC_BLK
    return pl.pallas_call(
        _tc_body,
        grid=(ncols // TC_BLK,),
        in_specs=[pl.BlockSpec((R, TC_BLK), lambda i, off=off: (0, i + off))],
        out_specs=pl.BlockSpec((TC_BLK,), lambda i: (i,)),
        out_shape=jax.ShapeDtypeStruct((ncols,), jnp.int32),
    )(x)


@jax.jit
def kernel(x):
    if SC_COLS == 0:
        return _argmax_tc(x, 0, N).astype(jnp.int64)
    sc = _argmax_sc(x)
    tc = _argmax_tc(x, SC_COLS, N - SC_COLS)
    return jnp.concatenate([sc, tc]).astype(jnp.int64)


# speedup vs baseline: 3.3947x; 1.1916x over previous
import functools

import jax
import jax.numpy as jnp
from jax import lax
from jax.experimental import pallas as pl
from jax.experimental.pallas import tpu as pltpu
from jax.experimental.pallas import tpu_sc as plsc

R = 128
N = 32768
NC = 2
NS = 16
L = 16
NW = NC * NS
SC_COLS = 0
CW = max(SC_COLS, NW) // NW
RB = 32
NRB = R // RB
NG = CW // L
CHAIN = 8


def _block_reduce(buf, col, row0, m_in, i_in):
    chains = []
    for c in range(RB // CHAIN):
        r = c * CHAIN
        m = buf[r, col]
        idx = jnp.full((L,), row0 + r, jnp.int32)
        for i in range(r + 1, r + CHAIN):
            v = buf[i, col]
            p = v > m
            m = jnp.where(p, v, m)
            idx = jnp.where(p, row0 + i, idx)
        chains.append((m, idx))
    while len(chains) > 1:
        nxt = []
        for a in range(0, len(chains), 2):
            (ma, ia), (mb, ib) = chains[a], chains[a + 1]
            p = mb > ma
            nxt.append((jnp.where(p, mb, ma), jnp.where(p, ib, ia)))
        chains = nxt
    m, idx = chains[0]
    if m_in is not None:
        p = m > m_in
        m = jnp.where(p, m, m_in)
        idx = jnp.where(p, idx, i_in)
    return m, idx


def _argmax_body(x_hbm, out_hbm, buf0, buf1, mscr, iscr, sem0, sem1, osem):
    wid = lax.axis_index("s") * NC + lax.axis_index("c")
    base = wid * CW
    bufs = (buf0, buf1)
    sems = (sem0, sem1)

    def fetch(rb):
        return pltpu.async_copy(
            x_hbm.at[pl.ds(rb * RB, RB), pl.ds(base, CW)], bufs[rb % 2],
            sems[rb % 2])

    dma = fetch(0)
    for rb in range(NRB):
        nxt = fetch(rb + 1) if rb + 1 < NRB else None
        dma.wait()
        buf = bufs[rb % 2]
        first = rb == 0

        def gbody(g, _, buf=buf, first=first, rb=rb):
            col = pl.ds(g * L, L)
            if first:
                m_in, i_in = None, None
            else:
                m_in, i_in = mscr[col], iscr[col]
            m, idx = _block_reduce(buf, col, rb * RB, m_in, i_in)
            mscr[col] = m
            iscr[col] = idx
            return 0

        lax.fori_loop(0, NG, gbody, 0)
        dma = nxt

    pltpu.async_copy(iscr, out_hbm.at[pl.ds(base, CW)], osem).wait()


if SC_COLS:
    _argmax_sc = functools.partial(
        pl.kernel,
        out_type=jax.ShapeDtypeStruct((SC_COLS,), jnp.int32),
        mesh=plsc.VectorSubcoreMesh(core_axis_name="c", subcore_axis_name="s",
                                    num_cores=NC, num_subcores=NS),
        scratch_types=[
            pltpu.VMEM((RB, CW), jnp.float32),
            pltpu.VMEM((RB, CW), jnp.float32),
            pltpu.VMEM((CW,), jnp.float32),
            pltpu.VMEM((CW,), jnp.int32),
            pltpu.SemaphoreType.DMA,
            pltpu.SemaphoreType.DMA,
            pltpu.SemaphoreType.DMA,
        ],
    )(_argmax_body)


TC_BLK = 8192


def _tc_body(x_ref, o_ref):
    xb = x_ref[...]
    m = jnp.max(xb, axis=0)
    rowid = lax.broadcasted_iota(jnp.int32, xb.shape, 0)
    cand = jnp.where(xb == m[None, :], rowid, R)
    o_ref[...] = jnp.min(cand, axis=0)


def _argmax_tc(x, start, ncols):
    off = start // TC_BLK
    return pl.pallas_call(
        _tc_body,
        grid=(ncols // TC_BLK,),
        in_specs=[pl.BlockSpec((R, TC_BLK), lambda i, off=off: (0, i + off))],
        out_specs=pl.BlockSpec((TC_BLK,), lambda i: (i,)),
        out_shape=jax.ShapeDtypeStruct((ncols,), jnp.int32),
    )(x)


@jax.jit
def kernel(x):
    if SC_COLS == 0:
        return _argmax_tc(x, 0, N).astype(jnp.int64)
    sc = _argmax_sc(x)
    tc = _argmax_tc(x, SC_COLS, N - SC_COLS)
    return jnp.concatenate([sc, tc]).astype(jnp.int64)


# speedup vs baseline: 3.4206x; 1.0076x over previous
import functools

import jax
import jax.numpy as jnp
from jax import lax
from jax.experimental import pallas as pl
from jax.experimental.pallas import tpu as pltpu
from jax.experimental.pallas import tpu_sc as plsc

R = 128
N = 32768
NC = 2
NS = 16
L = 16
NW = NC * NS
SC_COLS = 0
CW = max(SC_COLS, NW) // NW
RB = 32
NRB = R // RB
NG = CW // L
CHAIN = 8


def _block_reduce(buf, col, row0, m_in, i_in):
    chains = []
    for c in range(RB // CHAIN):
        r = c * CHAIN
        m = buf[r, col]
        idx = jnp.full((L,), row0 + r, jnp.int32)
        for i in range(r + 1, r + CHAIN):
            v = buf[i, col]
            p = v > m
            m = jnp.where(p, v, m)
            idx = jnp.where(p, row0 + i, idx)
        chains.append((m, idx))
    while len(chains) > 1:
        nxt = []
        for a in range(0, len(chains), 2):
            (ma, ia), (mb, ib) = chains[a], chains[a + 1]
            p = mb > ma
            nxt.append((jnp.where(p, mb, ma), jnp.where(p, ib, ia)))
        chains = nxt
    m, idx = chains[0]
    if m_in is not None:
        p = m > m_in
        m = jnp.where(p, m, m_in)
        idx = jnp.where(p, idx, i_in)
    return m, idx


def _argmax_body(x_hbm, out_hbm, buf0, buf1, mscr, iscr, sem0, sem1, osem):
    wid = lax.axis_index("s") * NC + lax.axis_index("c")
    base = wid * CW
    bufs = (buf0, buf1)
    sems = (sem0, sem1)

    def fetch(rb):
        return pltpu.async_copy(
            x_hbm.at[pl.ds(rb * RB, RB), pl.ds(base, CW)], bufs[rb % 2],
            sems[rb % 2])

    dma = fetch(0)
    for rb in range(NRB):
        nxt = fetch(rb + 1) if rb + 1 < NRB else None
        dma.wait()
        buf = bufs[rb % 2]
        first = rb == 0

        def gbody(g, _, buf=buf, first=first, rb=rb):
            col = pl.ds(g * L, L)
            if first:
                m_in, i_in = None, None
            else:
                m_in, i_in = mscr[col], iscr[col]
            m, idx = _block_reduce(buf, col, rb * RB, m_in, i_in)
            mscr[col] = m
            iscr[col] = idx
            return 0

        lax.fori_loop(0, NG, gbody, 0)
        dma = nxt

    pltpu.async_copy(iscr, out_hbm.at[pl.ds(base, CW)], osem).wait()


if SC_COLS:
    _argmax_sc = functools.partial(
        pl.kernel,
        out_type=jax.ShapeDtypeStruct((SC_COLS,), jnp.int32),
        mesh=plsc.VectorSubcoreMesh(core_axis_name="c", subcore_axis_name="s",
                                    num_cores=NC, num_subcores=NS),
        scratch_types=[
            pltpu.VMEM((RB, CW), jnp.float32),
            pltpu.VMEM((RB, CW), jnp.float32),
            pltpu.VMEM((CW,), jnp.float32),
            pltpu.VMEM((CW,), jnp.int32),
            pltpu.SemaphoreType.DMA,
            pltpu.SemaphoreType.DMA,
            pltpu.SemaphoreType.DMA,
        ],
    )(_argmax_body)


TC_BLK = 16384


def _tc_body(x_ref, o_ref):
    xb = x_ref[...]
    m = jnp.max(xb, axis=0)
    rowid = lax.broadcasted_iota(jnp.int32, xb.shape, 0)
    cand = jnp.where(xb == m[None, :], rowid, R)
    o_ref[...] = jnp.min(cand, axis=0)


def _argmax_tc(x, start, ncols):
    off = start // TC_BLK
    return pl.pallas_call(
        _tc_body,
        grid=(ncols // TC_BLK,),
        in_specs=[pl.BlockSpec((R, TC_BLK), lambda i, off=off: (0, i + off))],
        out_specs=pl.BlockSpec((TC_BLK,), lambda i: (i,)),
        out_shape=jax.ShapeDtypeStruct((ncols,), jnp.int32),
    )(x)


@jax.jit
def kernel(x):
    if SC_COLS == 0:
        return _argmax_tc(x, 0, N).astype(jnp.int64)
    sc = _argmax_sc(x)
    tc = _argmax_tc(x, SC_COLS, N - SC_COLS)
    return jnp.concatenate([sc, tc]).astype(jnp.int64)
